# baseline (speedup 1.0000x reference)
import functools

import jax
import jax.numpy as jnp
from jax import lax
from jax.experimental import pallas as pl
from jax.experimental.pallas import tpu as pltpu
from jax.experimental.pallas import tpu_sc as plsc

FEAT = 128
NC, NS = 2, 16
NW = NC * NS
GCHUNK = 128


def _sc_gather(masks, idx):
    B = idx.shape[0]
    per_w = B // NW
    n_g = per_w // GCHUNK

    mesh = plsc.VectorSubcoreMesh(core_axis_name="c", subcore_axis_name="s")

    @functools.partial(
        pl.kernel, mesh=mesh,
        out_type=jax.ShapeDtypeStruct((B, FEAT), jnp.bool_),
        scratch_types=[
            pltpu.VMEM((n_g, GCHUNK), jnp.int32),
            pltpu.VMEM((per_w, FEAT), jnp.bool_),
            pltpu.SemaphoreType.DMA,
        ],
    )
    def k(masks_hbm, idx_hbm, out_hbm, idx_v, rows_v, sem):
        wid = lax.axis_index("s") * NC + lax.axis_index("c")
        base = wid * per_w
        pltpu.sync_copy(idx_hbm.at[wid], idx_v)
        for g in range(n_g):
            pltpu.async_copy(
                masks_hbm.at[idx_v.at[g]],
                rows_v.at[pl.ds(g * GCHUNK, GCHUNK)],
                sem,
            )
        for g in range(n_g):
            pltpu.make_async_copy(
                masks_hbm.at[idx_v.at[g]],
                rows_v.at[pl.ds(g * GCHUNK, GCHUNK)],
                sem,
            ).wait()
        pltpu.sync_copy(rows_v, out_hbm.at[pl.ds(base, per_w)])

    return k(masks, idx.reshape(NW, n_g, GCHUNK))


def _tc_fill_body(z_ref, m_ref, o_ref):
    o_ref[...] = jnp.where(m_ref[...], jnp.zeros((), jnp.float32), z_ref[...])


def _tc_fill(z, mask):
    B = z.shape[0]
    blk = 2048
    return pl.pallas_call(
        _tc_fill_body,
        grid=(B // blk,),
        in_specs=[
            pl.BlockSpec((blk, FEAT), lambda i: (i, 0)),
            pl.BlockSpec((blk, FEAT), lambda i: (i, 0)),
        ],
        out_specs=pl.BlockSpec((blk, FEAT), lambda i: (i, 0)),
        out_shape=jax.ShapeDtypeStruct((B, FEAT), jnp.float32),
    )(z, mask)


def kernel(z, idx, masks):
    mask = _sc_gather(masks, idx.astype(jnp.int32))
    z_masked = _tc_fill(z, mask)
    return (z_masked, mask)
